# baseline (speedup 1.0000x reference)
import dataclasses
import functools

import jax
import jax.numpy as jnp
import numpy as np
from jax import lax
from jax.experimental import pallas as pl
from jax.experimental.pallas import tpu as pltpu
from jax.experimental.pallas import tpu_sc as plsc

_L = 16
_T = 2 ** 19
_F = 2
_NUM_SAMPLE = 64
_NEAR = 2.0
_FAR = 6.0
_BB_MIN = -8.0
_B_GROWTH = np.exp((np.log(2048.0) - np.log(16.0)) / (_L - 1))
_RES = [int(np.floor(16 * _B_GROWTH ** i)) for i in range(_L)]
_PI2 = int(np.uint32(2654435761).astype(np.int32))
_PI3 = 805459861
_MASK = _T - 1

_NUM_RAYS = 4096
_NUM_TILES = 32
_RAYS_PER_TILE = _NUM_RAYS // _NUM_TILES
_PTS_PER_TILE = _RAYS_PER_TILE * _NUM_SAMPLE
_CHUNK_PTS = 128
_CHUNK_RAYS = _CHUNK_PTS // _NUM_SAMPLE
_N_CHUNKS = _PTS_PER_TILE // _CHUNK_PTS
_N_GROUPS = _CHUNK_PTS // 16
_DT = (_FAR - _NEAR) / (_NUM_SAMPLE - 1)
_OUT_D = 2 * _L + 16

_C0 = 0.28209479177387814
_C1 = 0.4886025119029199
_C2 = [1.0925484305920792, -1.0925484305920792, 0.31539156525252005,
       -1.0925484305920792, 0.5462742152960396]
_C3 = [-0.5900435899266435, 2.890611442640554, -0.4570457994644658,
       0.3731763325901154, -0.4570457994644658, 1.445305721320277,
       -0.5900435899266435]


def _sh_comps(x, y, z):
    xx, yy, zz = x * x, y * y, z * z
    xy, yz_, xz = x * y, y * z, x * z
    return [
        jnp.full_like(x, _C0),
        -_C1 * y, _C1 * z, -_C1 * x,
        _C2[0] * xy, _C2[1] * yz_, _C2[2] * (2.0 * zz - xx - yy),
        _C2[3] * xz, _C2[4] * (xx - yy),
        _C3[0] * y * (3 * xx - yy), _C3[1] * xy * z,
        _C3[2] * y * (4 * zz - xx - yy),
        _C3[3] * z * (2 * zz - 3 * xx - 3 * yy),
        _C3[4] * x * (4 * zz - xx - yy),
        _C3[5] * z * (xx - yy), _C3[6] * x * (xx - 3 * yy),
    ]


def _sc_body(rays_hbm, tab_hbm, out_hbm, rays_v, sh_v, idx_v, rows_v, w_v,
             out_v, dma_sem):
    wid = lax.axis_index("s") * 2 + lax.axis_index("c")
    rbase = wid * _RAYS_PER_TILE
    pbase = wid * _PTS_PER_TILE

    pltpu.sync_copy(rays_hbm.at[pl.ds(rbase, _RAYS_PER_TILE)], rays_v)

    iota = lax.iota(jnp.int32, 16)
    fiota = iota.astype(jnp.float32)

    def full_i(v):
        return jnp.full((16,), v, jnp.int32)

    @pl.loop(0, _RAYS_PER_TILE // 16)
    def _(rg):
        rows = rg * 16 + iota
        x = plsc.load_gather(rays_v, [rows, full_i(0)])
        y = plsc.load_gather(rays_v, [rows, full_i(1)])
        z = plsc.load_gather(rays_v, [rows, full_i(2)])
        for k, v in enumerate(_sh_comps(x, y, z)):
            plsc.store_scatter(sh_v, [rows, full_i(k)], v)

    @pl.loop(0, _N_CHUNKS)
    def _(chunk):
        @pl.loop(0, _N_GROUPS)
        def _(g):
            ray = chunk * _CHUNK_RAYS + g // 4
            rvec = jnp.full((16,), ray, jnp.int32)
            col = g * 16
            tvec = jnp.float32(_NEAR) + ((g % 4) * 16 + iota).astype(
                jnp.float32) * jnp.float32(_DT)
            x = (plsc.load_gather(rays_v, [rvec, full_i(3)])
                 + plsc.load_gather(rays_v, [rvec, full_i(0)]) * tvec)
            y = (plsc.load_gather(rays_v, [rvec, full_i(4)])
                 + plsc.load_gather(rays_v, [rvec, full_i(1)]) * tvec)
            z = (plsc.load_gather(rays_v, [rvec, full_i(5)])
                 + plsc.load_gather(rays_v, [rvec, full_i(2)]) * tvec)
            for l in range(_L):
                inv_cell = jnp.float32(_RES[l] / 16.0)
                ux = (x - jnp.float32(_BB_MIN)) * inv_cell
                uy = (y - jnp.float32(_BB_MIN)) * inv_cell
                uz = (z - jnp.float32(_BB_MIN)) * inv_cell
                xi = ux.astype(jnp.int32)
                yi = uy.astype(jnp.int32)
                zi = uz.astype(jnp.int32)
                w_v[0, l, pl.ds(col, 16)] = ux - xi.astype(jnp.float32)
                w_v[1, l, pl.ds(col, 16)] = uy - yi.astype(jnp.float32)
                w_v[2, l, pl.ds(col, 16)] = uz - zi.astype(jnp.float32)
                hx = (xi, xi + 1)
                hy0 = yi * _PI2
                hy = (hy0, hy0 + _PI2)
                hz0 = zi * _PI3
                hz = (hz0, hz0 + _PI3)
                base = l * _T
                for c in range(8):
                    h = hx[c >> 2] ^ hy[(c >> 1) & 1] ^ hz[c & 1]
                    idx_v[pl.ds((l * 8 + c) * _CHUNK_PTS + col, 16)] = (
                        (h & _MASK) + base)

        def fire_level(l, buf):
            pltpu.async_copy(
                tab_hbm.at[idx_v.at[pl.ds(l * 8 * _CHUNK_PTS,
                                          8 * _CHUNK_PTS)]],
                rows_v.at[buf], dma_sem)

        def drain_level(l, buf):
            pltpu.make_async_copy(
                tab_hbm.at[idx_v.at[pl.ds(l * 8 * _CHUNK_PTS,
                                          8 * _CHUNK_PTS)]],
                rows_v.at[buf], dma_sem).wait()

        fire_level(0, 0)

        @pl.loop(0, _L)
        def _(l):
            buf = l & 1
            drain_level(l, buf)

            @pl.when(l < _L - 1)
            def _():
                fire_level(l + 1, 1 - buf)

            @pl.loop(0, _N_GROUPS)
            def _(g):
                col = g * 16
                pvec = col + iota
                fx = w_v[0, l, pl.ds(col, 16)]
                fy = w_v[1, l, pl.ds(col, 16)]
                fz = w_v[2, l, pl.ds(col, 16)]
                gx, gy, gz = 1.0 - fx, 1.0 - fy, 1.0 - fz
                e = []
                for c in range(8):
                    rvec = c * _CHUNK_PTS + pvec
                    e.append((plsc.load_gather(rows_v,
                                               [full_i(buf), rvec,
                                                full_i(0)]),
                              plsc.load_gather(rows_v,
                                               [full_i(buf), rvec,
                                                full_i(1)])))
                for f in range(2):
                    c00 = e[0][f] * gx + e[4][f] * fx
                    c01 = e[1][f] * gx + e[5][f] * fx
                    c10 = e[2][f] * gx + e[6][f] * fx
                    c11 = e[3][f] * gx + e[7][f] * fx
                    c0 = c00 * gy + c10 * fy
                    c1 = c01 * gy + c11 * fy
                    plsc.store_scatter(out_v, [pvec, full_i(2 * l + f)],
                                       c0 * gz + c1 * fz)

        @pl.loop(0, _CHUNK_PTS)
        def _(p):
            ray = chunk * _CHUNK_RAYS + p // _NUM_SAMPLE
            out_v[p, pl.ds(2 * _L, 16)] = sh_v[ray, pl.ds(0, 16)]

        pltpu.sync_copy(
            out_v, out_hbm.at[pl.ds(pbase + chunk * _CHUNK_PTS, _CHUNK_PTS)])


def _compiler_params():
    cp = pltpu.CompilerParams()
    for field, val in (("needs_layout_passes", False),
                       ("use_tc_tiling_on_sc", False)):
        if field in pltpu.CompilerParams.__dataclass_fields__:
            cp = dataclasses.replace(cp, **{field: val})
    return cp


@functools.partial(jax.jit)
def kernel(rays, tables):
    tab = tables.reshape(_L * _T, _F)
    run = pl.kernel(
        _sc_body,
        out_type=jax.ShapeDtypeStruct((_NUM_RAYS * _NUM_SAMPLE, _OUT_D),
                                      jnp.float32),
        mesh=plsc.VectorSubcoreMesh(core_axis_name="c", subcore_axis_name="s"),
        compiler_params=_compiler_params(),
        scratch_types=[
            pltpu.VMEM((_RAYS_PER_TILE, 6), jnp.float32),
            pltpu.VMEM((_RAYS_PER_TILE, 16), jnp.float32),
            pltpu.VMEM((_L * 8 * _CHUNK_PTS,), jnp.int32),
            pltpu.VMEM((2, 8 * _CHUNK_PTS, _F), jnp.float32),
            pltpu.VMEM((3, _L, _CHUNK_PTS), jnp.float32),
            pltpu.VMEM((_CHUNK_PTS, _OUT_D), jnp.float32),
            pltpu.SemaphoreType.DMA,
        ],
    )
    out = run(rays, tab)
    return out.reshape(_NUM_RAYS, _NUM_SAMPLE, _OUT_D)


# speedup vs baseline: 1.0820x; 1.0820x over previous
import dataclasses
import functools

import jax
import jax.numpy as jnp
import numpy as np
from jax import lax
from jax.experimental import pallas as pl
from jax.experimental.pallas import tpu as pltpu
from jax.experimental.pallas import tpu_sc as plsc

_L = 16
_T = 2 ** 19
_F = 2
_NUM_SAMPLE = 64
_NEAR = 2.0
_FAR = 6.0
_BB_MIN = -8.0
_B_GROWTH = np.exp((np.log(2048.0) - np.log(16.0)) / (_L - 1))
_RES = [int(np.floor(16 * _B_GROWTH ** i)) for i in range(_L)]
_PI2 = int(np.uint32(2654435761).astype(np.int32))
_PI3 = 805459861
_MASK = _T - 1

_NUM_RAYS = 4096
_NUM_TILES = 32
_RAYS_PER_TILE = _NUM_RAYS // _NUM_TILES
_PTS_PER_TILE = _RAYS_PER_TILE * _NUM_SAMPLE
_CHUNK_PTS = 128
_CHUNK_RAYS = _CHUNK_PTS // _NUM_SAMPLE
_N_CHUNKS = _PTS_PER_TILE // _CHUNK_PTS
_N_GROUPS = _CHUNK_PTS // 16
_DT = (_FAR - _NEAR) / (_NUM_SAMPLE - 1)
_OUT_D = 2 * _L + 16
_LVL_ELEMS = 16 * _CHUNK_PTS

_C0 = 0.28209479177387814
_C1 = 0.4886025119029199
_C2 = [1.0925484305920792, -1.0925484305920792, 0.31539156525252005,
       -1.0925484305920792, 0.5462742152960396]
_C3 = [-0.5900435899266435, 2.890611442640554, -0.4570457994644658,
       0.3731763325901154, -0.4570457994644658, 1.445305721320277,
       -0.5900435899266435]


def _sh_comps(x, y, z):
    xx, yy, zz = x * x, y * y, z * z
    xy, yz_, xz = x * y, y * z, x * z
    return [
        jnp.full_like(x, _C0),
        -_C1 * y, _C1 * z, -_C1 * x,
        _C2[0] * xy, _C2[1] * yz_, _C2[2] * (2.0 * zz - xx - yy),
        _C2[3] * xz, _C2[4] * (xx - yy),
        _C3[0] * y * (3 * xx - yy), _C3[1] * xy * z,
        _C3[2] * y * (4 * zz - xx - yy),
        _C3[3] * z * (2 * zz - 3 * xx - 3 * yy),
        _C3[4] * x * (4 * zz - xx - yy),
        _C3[5] * z * (xx - yy), _C3[6] * x * (xx - 3 * yy),
    ]


def _sc_body(rays_hbm, tab_hbm, out_hbm, rays_v, sh_v, idx_v, rows_v, w_v,
             out_v, dma_sem):
    wid = lax.axis_index("s") * 2 + lax.axis_index("c")
    rbase = wid * _RAYS_PER_TILE
    pbase = wid * _PTS_PER_TILE

    pltpu.sync_copy(rays_hbm.at[pl.ds(rbase * 6, _RAYS_PER_TILE * 6)], rays_v)

    iota = lax.iota(jnp.int32, 16)

    def full_i(v):
        return jnp.full((16,), v, jnp.int32)

    @pl.loop(0, _RAYS_PER_TILE // 16)
    def _(rg):
        rows = rg * 16 + iota
        rb = rows * 6
        x = plsc.load_gather(rays_v, [rb])
        y = plsc.load_gather(rays_v, [rb + 1])
        z = plsc.load_gather(rays_v, [rb + 2])
        for k, v in enumerate(_sh_comps(x, y, z)):
            plsc.store_scatter(sh_v, [rows, full_i(k)], v)

    @pl.loop(0, _N_CHUNKS)
    def _(chunk):
        @pl.loop(0, _N_GROUPS)
        def _(g):
            rb = full_i((chunk * _CHUNK_RAYS + g // 4) * 6)
            col = g * 16
            tvec = jnp.float32(_NEAR) + ((g % 4) * 16 + iota).astype(
                jnp.float32) * jnp.float32(_DT)
            x = (plsc.load_gather(rays_v, [rb + 3])
                 + plsc.load_gather(rays_v, [rb]) * tvec)
            y = (plsc.load_gather(rays_v, [rb + 4])
                 + plsc.load_gather(rays_v, [rb + 1]) * tvec)
            z = (plsc.load_gather(rays_v, [rb + 5])
                 + plsc.load_gather(rays_v, [rb + 2]) * tvec)
            for l in range(_L):
                inv_cell = jnp.float32(_RES[l] / 16.0)
                ux = (x - jnp.float32(_BB_MIN)) * inv_cell
                uy = (y - jnp.float32(_BB_MIN)) * inv_cell
                uz = (z - jnp.float32(_BB_MIN)) * inv_cell
                xi = ux.astype(jnp.int32)
                yi = uy.astype(jnp.int32)
                zi = uz.astype(jnp.int32)
                w_v[0, l, pl.ds(col, 16)] = ux - xi.astype(jnp.float32)
                w_v[1, l, pl.ds(col, 16)] = uy - yi.astype(jnp.float32)
                w_v[2, l, pl.ds(col, 16)] = uz - zi.astype(jnp.float32)
                hx = (xi, xi + 1)
                hy0 = yi * _PI2
                hy = (hy0, hy0 + _PI2)
                hz0 = zi * _PI3
                hz = (hz0, hz0 + _PI3)
                base = l * _T
                for c in range(8):
                    h = hx[c >> 2] ^ hy[(c >> 1) & 1] ^ hz[c & 1]
                    ev = ((h & _MASK) + base) * 2
                    blk = (l * 16 + c * 2) * _CHUNK_PTS + col
                    idx_v[pl.ds(blk, 16)] = ev
                    idx_v[pl.ds(blk + _CHUNK_PTS, 16)] = ev + 1

        def fire_level(l, buf):
            pltpu.async_copy(
                tab_hbm.at[idx_v.at[pl.ds(l * _LVL_ELEMS, _LVL_ELEMS)]],
                rows_v.at[buf], dma_sem)

        def drain_level(l, buf):
            pltpu.make_async_copy(
                tab_hbm.at[idx_v.at[pl.ds(l * _LVL_ELEMS, _LVL_ELEMS)]],
                rows_v.at[buf], dma_sem).wait()

        fire_level(0, 0)

        @pl.loop(0, _L)
        def _(l):
            buf = l & 1
            drain_level(l, buf)

            @pl.when(l < _L - 1)
            def _():
                fire_level(l + 1, 1 - buf)

            @pl.loop(0, _N_GROUPS)
            def _(g):
                col = g * 16
                pvec = col + iota
                fx = w_v[0, l, pl.ds(col, 16)]
                fy = w_v[1, l, pl.ds(col, 16)]
                fz = w_v[2, l, pl.ds(col, 16)]
                gx, gy, gz = 1.0 - fx, 1.0 - fy, 1.0 - fz
                bvec = full_i(buf)
                e = []
                for c in range(8):
                    off = c * 2 * _CHUNK_PTS + pvec
                    e.append((plsc.load_gather(rows_v, [bvec, off]),
                              plsc.load_gather(rows_v,
                                               [bvec, off + _CHUNK_PTS])))
                for f in range(2):
                    c00 = e[0][f] * gx + e[4][f] * fx
                    c01 = e[1][f] * gx + e[5][f] * fx
                    c10 = e[2][f] * gx + e[6][f] * fx
                    c11 = e[3][f] * gx + e[7][f] * fx
                    c0 = c00 * gy + c10 * fy
                    c1 = c01 * gy + c11 * fy
                    plsc.store_scatter(out_v, [pvec, full_i(2 * l + f)],
                                       c0 * gz + c1 * fz)

        @pl.loop(0, _CHUNK_PTS)
        def _(p):
            ray = chunk * _CHUNK_RAYS + p // _NUM_SAMPLE
            out_v[p, pl.ds(2 * _L, 16)] = sh_v[ray, pl.ds(0, 16)]

        pltpu.sync_copy(
            out_v, out_hbm.at[pl.ds(pbase + chunk * _CHUNK_PTS, _CHUNK_PTS)])


def _compiler_params():
    cp = pltpu.CompilerParams()
    for field, val in (("needs_layout_passes", False),
                       ("use_tc_tiling_on_sc", False)):
        if field in pltpu.CompilerParams.__dataclass_fields__:
            cp = dataclasses.replace(cp, **{field: val})
    return cp


@functools.partial(jax.jit)
def kernel(rays, tables):
    tab = tables.reshape(_L * _T * _F)
    rays_flat = rays.reshape(_NUM_RAYS * 6)
    run = pl.kernel(
        _sc_body,
        out_type=jax.ShapeDtypeStruct((_NUM_RAYS * _NUM_SAMPLE, _OUT_D),
                                      jnp.float32),
        mesh=plsc.VectorSubcoreMesh(core_axis_name="c", subcore_axis_name="s"),
        compiler_params=_compiler_params(),
        scratch_types=[
            pltpu.VMEM((_RAYS_PER_TILE * 6,), jnp.float32),
            pltpu.VMEM((_RAYS_PER_TILE, 16), jnp.float32),
            pltpu.VMEM((_L * 16 * _CHUNK_PTS,), jnp.int32),
            pltpu.VMEM((2, _LVL_ELEMS), jnp.float32),
            pltpu.VMEM((3, _L, _CHUNK_PTS), jnp.float32),
            pltpu.VMEM((_CHUNK_PTS, _OUT_D), jnp.float32),
            pltpu.SemaphoreType.DMA,
        ],
    )
    out = run(rays_flat, tab)
    return out.reshape(_NUM_RAYS, _NUM_SAMPLE, _OUT_D)


# speedup vs baseline: 1.3045x; 1.2056x over previous
import dataclasses
import functools

import jax
import jax.numpy as jnp
import numpy as np
from jax import lax
from jax.experimental import pallas as pl
from jax.experimental.pallas import tpu as pltpu
from jax.experimental.pallas import tpu_sc as plsc

_L = 16
_T = 2 ** 19
_F = 2
_NUM_SAMPLE = 64
_NEAR = 2.0
_FAR = 6.0
_BB_MIN = -8.0
_B_GROWTH = np.exp((np.log(2048.0) - np.log(16.0)) / (_L - 1))
_RES = [int(np.floor(16 * _B_GROWTH ** i)) for i in range(_L)]
_PI2 = int(np.uint32(2654435761).astype(np.int32))
_PI3 = 805459861
_MASK = _T - 1

_NUM_RAYS = 4096
_NUM_TILES = 32
_RAYS_PER_TILE = _NUM_RAYS // _NUM_TILES
_CHUNK_PTS = _RAYS_PER_TILE
_N_CHUNKS = _NUM_SAMPLE
_N_GROUPS = _CHUNK_PTS // 16
_DT = (_FAR - _NEAR) / (_NUM_SAMPLE - 1)
_OUT_D = 2 * _L + 16
_LVL_ELEMS = 8 * _CHUNK_PTS
_OUT_SPAN = 6 * 1024

_C0 = 0.28209479177387814
_C1 = 0.4886025119029199
_C2 = [1.0925484305920792, -1.0925484305920792, 0.31539156525252005,
       -1.0925484305920792, 0.5462742152960396]
_C3 = [-0.5900435899266435, 2.890611442640554, -0.4570457994644658,
       0.3731763325901154, -0.4570457994644658, 1.445305721320277,
       -0.5900435899266435]


def _sh_comps(x, y, z):
    xx, yy, zz = x * x, y * y, z * z
    xy, yz_, xz = x * y, y * z, x * z
    return [
        jnp.full_like(x, _C0),
        -_C1 * y, _C1 * z, -_C1 * x,
        _C2[0] * xy, _C2[1] * yz_, _C2[2] * (2.0 * zz - xx - yy),
        _C2[3] * xz, _C2[4] * (xx - yy),
        _C3[0] * y * (3 * xx - yy), _C3[1] * xy * z,
        _C3[2] * y * (4 * zz - xx - yy),
        _C3[3] * z * (2 * zz - 3 * xx - 3 * yy),
        _C3[4] * x * (4 * zz - xx - yy),
        _C3[5] * z * (xx - yy), _C3[6] * x * (xx - 3 * yy),
    ]


_RL_CHUNK = 16384
_RL_NCH = (_L * _T * _F // _NUM_TILES) // _RL_CHUNK


def _relayout_body(tabn_hbm, tabp_hbm, buf_v, obuf_v, in_sem, out_sem):
    wid = lax.axis_index("s") * 2 + lax.axis_index("c")
    base = wid * (_RL_NCH * _RL_CHUNK)
    iota = lax.iota(jnp.int32, 16)
    pat = (iota & 1) * 128 + (iota >> 1)

    def in_dma(ch, ib):
        return pltpu.make_async_copy(
            tabn_hbm.at[pl.ds(base + ch * _RL_CHUNK, _RL_CHUNK)],
            buf_v.at[ib], in_sem)

    def out_dma(ch, ib):
        return pltpu.make_async_copy(
            obuf_v.at[ib],
            tabp_hbm.at[pl.ds(base + ch * _RL_CHUNK, _RL_CHUNK)], out_sem)

    in_dma(0, 0).start()

    @pl.loop(0, _RL_NCH)
    def _(ch):
        ib = ch & 1
        in_dma(ch, ib).wait()

        @pl.when(ch < _RL_NCH - 1)
        def _():
            in_dma(ch + 1, 1 - ib).start()

        @pl.when(ch >= 2)
        def _():
            out_dma(ch - 2, ib).wait()

        ibv = jnp.full((16,), ib, jnp.int32)

        @pl.loop(0, _RL_CHUNK // 256)
        def _(b):
            for c in range(16):
                v = plsc.load_gather(buf_v, [ibv, b * 256 + c * 8 + pat])
                obuf_v[ib, pl.ds(b * 256 + c * 16, 16)] = v

        out_dma(ch, ib).start()

    @pl.loop(_RL_NCH - 2, _RL_NCH)
    def _(ch):
        out_dma(ch, ch & 1).wait()


def _sc_body(rays_hbm, tab_hbm, out_hbm, rays_v, idx_v, rows_v, w_v,
             out_v, dma_sem, out_sem):
    wid = lax.axis_index("s") * 2 + lax.axis_index("c")
    rbase = wid * _RAYS_PER_TILE

    pltpu.sync_copy(rays_hbm.at[pl.ds(rbase * 6, _RAYS_PER_TILE * 6)], rays_v)

    iota = lax.iota(jnp.int32, 16)

    @pl.loop(0, _N_GROUPS)
    def _(rg):
        rb = (rg * 16 + iota) * 6
        x = plsc.load_gather(rays_v, [rb])
        y = plsc.load_gather(rays_v, [rb + 1])
        z = plsc.load_gather(rays_v, [rb + 2])
        for k, v in enumerate(_sh_comps(x, y, z)):
            off = (k & 7) * 128 + rg * 16
            out_v[0, 4 + (k >> 3), pl.ds(off, 16)] = v
            out_v[1, 4 + (k >> 3), pl.ds(off, 16)] = v

    def out_dmas(chunk, obuf):
        return [
            pltpu.make_async_copy(
                out_v.at[obuf, cb],
                out_hbm.at[pl.ds(((chunk * 6 + cb) * _NUM_TILES + wid) * 1024,
                                 1024)],
                out_sem)
            for cb in range(6)
        ]

    @pl.loop(0, _N_CHUNKS)
    def _(chunk):
        obuf = chunk & 1

        @pl.when(chunk >= 2)
        def _():
            for d in out_dmas(chunk - 2, obuf):
                d.wait()

        tvec = (jnp.full((16,), chunk, jnp.int32).astype(jnp.float32)
                * jnp.float32(_DT) + jnp.float32(_NEAR))

        @pl.loop(0, _N_GROUPS)
        def _(g):
            col = g * 16
            rb = (col + iota) * 6
            x = (plsc.load_gather(rays_v, [rb + 3])
                 + plsc.load_gather(rays_v, [rb]) * tvec)
            y = (plsc.load_gather(rays_v, [rb + 4])
                 + plsc.load_gather(rays_v, [rb + 1]) * tvec)
            z = (plsc.load_gather(rays_v, [rb + 5])
                 + plsc.load_gather(rays_v, [rb + 2]) * tvec)
            for l in range(_L):
                inv_cell = jnp.float32(_RES[l] / 16.0)
                ux = (x - jnp.float32(_BB_MIN)) * inv_cell
                uy = (y - jnp.float32(_BB_MIN)) * inv_cell
                uz = (z - jnp.float32(_BB_MIN)) * inv_cell
                xi = ux.astype(jnp.int32)
                yi = uy.astype(jnp.int32)
                zi = uz.astype(jnp.int32)
                w_v[0, l, pl.ds(col, 16)] = ux - xi.astype(jnp.float32)
                w_v[1, l, pl.ds(col, 16)] = uy - yi.astype(jnp.float32)
                w_v[2, l, pl.ds(col, 16)] = uz - zi.astype(jnp.float32)
                hx = (xi, xi + 1)
                hy0 = yi * _PI2
                hy = (hy0, hy0 + _PI2)
                hz0 = zi * _PI3
                hz = (hz0, hz0 + _PI3)
                base = l * _T
                for c in range(8):
                    h = (hx[c >> 2] ^ hy[(c >> 1) & 1] ^ hz[c & 1]) & _MASK
                    idx_v[pl.ds((l * 8 + c) * _CHUNK_PTS + col, 16)] = h + base

        def fire_level(l, buf):
            pltpu.async_copy(
                tab_hbm.at[idx_v.at[pl.ds(l * _LVL_ELEMS, _LVL_ELEMS)]],
                rows_v.at[buf], dma_sem)

        def drain_level(l, buf):
            pltpu.make_async_copy(
                tab_hbm.at[idx_v.at[pl.ds(l * _LVL_ELEMS, _LVL_ELEMS)]],
                rows_v.at[buf], dma_sem).wait()

        fire_level(0, 0)

        @pl.loop(0, _L)
        def _(l):
            buf = l & 1
            drain_level(l, buf)

            @pl.when(l < _L - 1)
            def _():
                fire_level(l + 1, 1 - buf)

            @pl.loop(0, _N_GROUPS)
            def _(g):
                col = g * 16
                fx = w_v[0, l, pl.ds(col, 16)]
                fy = w_v[1, l, pl.ds(col, 16)]
                fz = w_v[2, l, pl.ds(col, 16)]
                gx, gy, gz = 1.0 - fx, 1.0 - fy, 1.0 - fz
                bvec = jnp.full((16,), buf, jnp.int32)
                f0 = jnp.zeros((16,), jnp.int32)
                f1 = jnp.full((16,), 1, jnp.int32)
                e = []
                for c in range(8):
                    rv = c * _CHUNK_PTS + col + iota
                    e.append((plsc.load_gather(rows_v, [bvec, rv, f0]),
                              plsc.load_gather(rows_v, [bvec, rv, f1])))
                for f in range(2):
                    c00 = e[0][f] * gx + e[4][f] * fx
                    c01 = e[1][f] * gx + e[5][f] * fx
                    c10 = e[2][f] * gx + e[6][f] * fx
                    c11 = e[3][f] * gx + e[7][f] * fx
                    c0 = c00 * gy + c10 * fy
                    c1 = c01 * gy + c11 * fy
                    ch = 2 * l + f
                    oout = (ch & 7) * 128 + col
                    out_v[obuf, ch >> 3, pl.ds(oout, 16)] = c0 * gz + c1 * fz

        for d in out_dmas(chunk, obuf):
            d.start()

    @pl.loop(_N_CHUNKS - 2, _N_CHUNKS)
    def _(chunk):
        for d in out_dmas(chunk, chunk & 1):
            d.wait()


def _compiler_params():
    cp = pltpu.CompilerParams()
    for field, val in (("needs_layout_passes", False),
                       ("use_tc_tiling_on_sc", False)):
        if field in pltpu.CompilerParams.__dataclass_fields__:
            cp = dataclasses.replace(cp, **{field: val})
    return cp


@functools.partial(jax.jit)
def kernel(rays, tables):
    tab = tables.reshape(_L, _T // 128, 128, _F).transpose(0, 1, 3, 2)
    tab = tab.reshape(_L * _T * _F)
    rays_flat = rays.reshape(_NUM_RAYS * 6)
    relayout = pl.kernel(
        _relayout_body,
        out_type=jax.ShapeDtypeStruct((_L * _T * _F,), jnp.float32),
        mesh=plsc.VectorSubcoreMesh(core_axis_name="c", subcore_axis_name="s"),
        compiler_params=_compiler_params(),
        scratch_types=[
            pltpu.VMEM((2, _RL_CHUNK), jnp.float32),
            pltpu.VMEM((2, _RL_CHUNK), jnp.float32),
            pltpu.SemaphoreType.DMA,
            pltpu.SemaphoreType.DMA,
        ],
    )
    tab_pairs = relayout(tab).reshape(_L * _T, _F)
    run = pl.kernel(
        _sc_body,
        out_type=jax.ShapeDtypeStruct(
            (_NUM_SAMPLE * _NUM_TILES * _OUT_SPAN,), jnp.float32),
        mesh=plsc.VectorSubcoreMesh(core_axis_name="c", subcore_axis_name="s"),
        compiler_params=_compiler_params(),
        scratch_types=[
            pltpu.VMEM((_RAYS_PER_TILE * 6,), jnp.float32),
            pltpu.VMEM((_L * 8 * _CHUNK_PTS,), jnp.int32),
            pltpu.VMEM((2, _LVL_ELEMS, _F), jnp.float32),
            pltpu.VMEM((3, _L, _CHUNK_PTS), jnp.float32),
            pltpu.VMEM((2, 6, 1024), jnp.float32),
            pltpu.SemaphoreType.DMA,
            pltpu.SemaphoreType.DMA,
        ],
    )
    out = run(rays_flat, tab_pairs)
    out = out.reshape(_NUM_SAMPLE, 6, _NUM_TILES, 8, 128)
    out = out.transpose(2, 4, 0, 1, 3)
    return out.reshape(_NUM_RAYS, _NUM_SAMPLE, _OUT_D)
